# baseline (speedup 1.0000x reference)
import jax
import jax.numpy as jnp
from jax import lax
from jax.experimental import pallas as pl
from jax.experimental.pallas import tpu as pltpu
from jax.experimental.pallas import tpu_sc as plsc

R, C = 32, 8192
N = R * C
K = N // 4
EPS = 1e-07
NT = 16
PER = N // NT
L = 16
NV = PER // L
NB = 128


def _loss_keys_body(pred_ref, targ_ref, key_ref):
    p = pred_ref[...]
    t = targ_ref[...]
    loss = jnp.maximum(p, 0.0) - p * t + jnp.log1p(jnp.exp(-jnp.abs(p)))
    bits = lax.bitcast_convert_type(loss, jnp.uint32)
    sign = bits >> jnp.uint32(31)
    flip = (jnp.uint32(0) - sign) | jnp.uint32(0x80000000)
    key_ref[...] = bits ^ flip


NW = NB * L


def _hist_pass(keys_v, hist_v, tmp_v, btot_sm, slots_hbm, sem, wid, shift,
               dmask, pshift, pref, r):
    zeros = jnp.zeros((L,), jnp.int32)
    ones = jnp.ones((L,), jnp.int32)

    @plsc.parallel_loop(0, NB, unroll=8)
    def _zero(j):
        hist_v[pl.ds(j * L, L)] = zeros

    @plsc.parallel_loop(0, NV, unroll=8)
    def _accum(i):
        kv = keys_v[pl.ds(i * L, L)]
        digit = lax.convert_element_type((kv >> jnp.uint32(shift))
                                         & jnp.uint32(dmask), jnp.int32)
        if pshift is None:
            plsc.addupdate_scatter(hist_v, [digit], ones)
        else:
            m = (kv >> jnp.uint32(pshift)) == jnp.uint32(pref)
            plsc.addupdate_scatter(hist_v, [digit], ones, mask=m)

    pltpu.sync_copy(hist_v, slots_hbm.at[pl.ds(wid * NW, NW)])
    plsc.subcore_barrier()
    hs = [pltpu.async_copy(slots_hbm.at[pl.ds(t * NW, NW)],
                           tmp_v.at[pl.ds(t * NW, NW)], sem)
          for t in range(NT)]
    for h in hs:
        h.wait()
    plsc.subcore_barrier()

    @plsc.parallel_loop(0, NB, unroll=2)
    def _merge(j):
        acc = tmp_v[pl.ds(j * L, L)]
        for t in range(1, NT):
            acc = acc + tmp_v[pl.ds(t * NW + j * L, L)]
        hist_v[pl.ds(j * L, L)] = acc
        btot_sm[j] = jnp.sum(acc)

    def _scana(i, carry):
        s, found, jstar, sbef = carry
        j = NB - 1 - i
        t = btot_sm[j]
        cross = (found == 0) & ((s + t) >= r)
        jstar = jnp.where(cross, j, jstar)
        sbef = jnp.where(cross, s, sbef)
        found = found | cross.astype(jnp.int32)
        return s + t, found, jstar, sbef

    zi = jnp.int32(0)
    _, _, jstar, sbef = lax.fori_loop(0, NB, _scana, (zi, zi, zi, zi))

    v = hist_v[pl.ds(jstar * L, L)]
    rv = lax.rev(v, (0,))
    cum = jnp.cumsum(rv)
    m = (sbef + cum) >= r
    pc = jnp.max(plsc.all_reduce_population_count(m))
    d_star = L * jstar + pc - 1
    above = sbef + jnp.sum(jnp.where(m, 0, rv))
    return d_star, above


def _sc_body(key_hbm, out_hbm, slots_hbm, pf_hbm, pi_hbm, keys_v, hist_v,
             tmp_v, vec_f, vec_i, gat_f, gat_i, btot_sm, sem):
    wid = lax.axis_index("s")
    base = wid * PER
    pltpu.sync_copy(key_hbm.at[pl.ds(base, PER)], keys_v)

    r = jnp.int32(K)
    d1, a1 = _hist_pass(keys_v, hist_v, tmp_v, btot_sm, slots_hbm, sem, wid,
                        24, 0xFF, None, None, r)
    r = r - a1
    p1 = d1
    d2, a2 = _hist_pass(keys_v, hist_v, tmp_v, btot_sm, slots_hbm, sem, wid,
                        16, 0xFF, 24, p1, r)
    r = r - a2
    p2 = (p1 << 8) | d2
    d3, a3 = _hist_pass(keys_v, hist_v, tmp_v, btot_sm, slots_hbm, sem, wid,
                        8, 0xFF, 16, p2, r)
    r = r - a3
    p3 = (p2 << 8) | d3
    d4, _ = _hist_pass(keys_v, hist_v, tmp_v, btot_sm, slots_hbm, sem, wid,
                       0, 0xFF, 8, p3, r)
    thresh = lax.convert_element_type((p3 << 8) | d4, jnp.uint32)

    U = 4

    def _reduce(i, carry):
        accs = list(carry)
        for u in range(U):
            kv = keys_v[pl.ds((i * U + u) * L, L)]
            sign = kv >> jnp.uint32(31)
            flip = jnp.uint32(0xFFFFFFFF) ^ (sign * jnp.uint32(0x7FFFFFFF))
            lossv = plsc.bitcast(kv ^ flip, jnp.float32)
            m = kv >= thresh
            accs[2 * u] = accs[2 * u] + jnp.where(m, lossv, 0.0)
            accs[2 * u + 1] = accs[2 * u + 1] + jnp.where(m, 1, 0)
        return tuple(accs)

    init = []
    for u in range(U):
        init.append(jnp.zeros((L,), jnp.float32))
        init.append(jnp.zeros((L,), jnp.int32))
    accs = lax.fori_loop(0, NV // U, _reduce, tuple(init))
    acc_s = accs[0] + accs[2] + accs[4] + accs[6]
    acc_c = accs[1] + accs[3] + accs[5] + accs[7]
    vec_f[...] = acc_s
    vec_i[...] = acc_c
    pltpu.sync_copy(vec_f, pf_hbm.at[wid])
    pltpu.sync_copy(vec_i, pi_hbm.at[wid])
    plsc.subcore_barrier()
    pltpu.sync_copy(pf_hbm, gat_f)
    pltpu.sync_copy(pi_hbm, gat_i)
    tot_f = jnp.zeros((L,), jnp.float32)
    tot_i = jnp.zeros((L,), jnp.int32)
    for t in range(NT):
        tot_f = tot_f + gat_f[t]
        tot_i = tot_i + gat_i[t]
    s = jnp.sum(tot_f)
    cnt = lax.convert_element_type(jnp.sum(tot_i), jnp.float32)
    res = jnp.broadcast_to(s, (L,)) / (jnp.broadcast_to(cnt, (L,)) + EPS)
    vec_f[...] = res

    @pl.when(wid == 0)
    def _():
        pltpu.sync_copy(vec_f, out_hbm)


_sc_select = pl.kernel(
    _sc_body,
    out_type=(jax.ShapeDtypeStruct((L,), jnp.float32),
              jax.ShapeDtypeStruct((NT * NB * L,), jnp.int32),
              jax.ShapeDtypeStruct((NT, L), jnp.float32),
              jax.ShapeDtypeStruct((NT, L), jnp.int32)),
    mesh=plsc.VectorSubcoreMesh(core_axis_name="c", subcore_axis_name="s",
                                num_cores=1),
    compiler_params=pltpu.CompilerParams(needs_layout_passes=False),
    scratch_types=[
        pltpu.VMEM((PER,), jnp.uint32),
        pltpu.VMEM((NB * L,), jnp.int32),
        pltpu.VMEM((NT * NB * L,), jnp.int32),
        pltpu.VMEM((L,), jnp.float32),
        pltpu.VMEM((L,), jnp.int32),
        pltpu.VMEM((NT, L), jnp.float32),
        pltpu.VMEM((NT, L), jnp.int32),
        pltpu.SMEM((NB,), jnp.int32),
        pltpu.SemaphoreType.DMA,
    ],
)


@jax.jit
def kernel(pred, target):
    keys = pl.pallas_call(
        _loss_keys_body,
        out_shape=jax.ShapeDtypeStruct((R, C), jnp.uint32),
    )(pred, target)
    out = _sc_select(keys.reshape(N))[0]
    return out[0]


# speedup vs baseline: 1.2271x; 1.2271x over previous
import jax
import jax.numpy as jnp
from jax import lax
from jax.experimental import pallas as pl
from jax.experimental.pallas import tpu as pltpu
from jax.experimental.pallas import tpu_sc as plsc

R, C = 32, 8192
N = R * C
K = N // 4
EPS = 1e-07
NT = 16
PER = N // NT
L = 16
NV = PER // L
NB = 128


def _loss_keys_body(pred_ref, targ_ref, key_ref):
    p = pred_ref[...]
    t = targ_ref[...]
    loss = jnp.maximum(p, 0.0) - p * t + jnp.log1p(jnp.exp(-jnp.abs(p)))
    bits = lax.bitcast_convert_type(loss, jnp.uint32)
    sign = bits >> jnp.uint32(31)
    flip = (jnp.uint32(0) - sign) | jnp.uint32(0x80000000)
    key_ref[...] = bits ^ flip


NW = NB * L


def _hist_pass(keys_v, hist_v, hist_b, tmp_v, btot_sm, slots_hbm, sem, wid,
               shift, dmask, pshift, pref, r):
    zeros = jnp.zeros((L,), jnp.int32)
    ones = jnp.ones((L,), jnp.int32)

    @plsc.parallel_loop(0, NB, unroll=8)
    def _zero(j):
        hist_v[pl.ds(j * L, L)] = zeros
        hist_b[pl.ds(j * L, L)] = zeros

    @plsc.parallel_loop(0, NV // 2, unroll=4)
    def _accum(i):
        for h, idx in ((hist_v, 2 * i), (hist_b, 2 * i + 1)):
            kv = keys_v[pl.ds(idx * L, L)]
            digit = lax.convert_element_type((kv >> jnp.uint32(shift))
                                             & jnp.uint32(dmask), jnp.int32)
            if pshift is None:
                plsc.addupdate_scatter(h, [digit], ones)
            else:
                m = (kv >> jnp.uint32(pshift)) == jnp.uint32(pref)
                plsc.addupdate_scatter(h, [digit], ones, mask=m)

    @plsc.parallel_loop(0, NB, unroll=8)
    def _comb(j):
        hist_v[pl.ds(j * L, L)] = (hist_v[pl.ds(j * L, L)]
                                   + hist_b[pl.ds(j * L, L)])

    pltpu.sync_copy(hist_v, slots_hbm.at[pl.ds(wid * NW, NW)])
    plsc.subcore_barrier()
    hs = [pltpu.async_copy(slots_hbm.at[pl.ds(t * NW, NW)],
                           tmp_v.at[pl.ds(t * NW, NW)], sem)
          for t in range(NT)]
    for h in hs:
        h.wait()
    plsc.subcore_barrier()

    @plsc.parallel_loop(0, NB, unroll=2)
    def _merge(j):
        acc = tmp_v[pl.ds(j * L, L)]
        for t in range(1, NT):
            acc = acc + tmp_v[pl.ds(t * NW + j * L, L)]
        hist_v[pl.ds(j * L, L)] = acc
        btot_sm[j] = jnp.sum(acc)

    def _scana(i, carry):
        s, found, jstar, sbef = carry
        j = NB - 1 - i
        t = btot_sm[j]
        cross = (found == 0) & ((s + t) >= r)
        jstar = jnp.where(cross, j, jstar)
        sbef = jnp.where(cross, s, sbef)
        found = found | cross.astype(jnp.int32)
        return s + t, found, jstar, sbef

    zi = jnp.int32(0)
    _, _, jstar, sbef = lax.fori_loop(0, NB, _scana, (zi, zi, zi, zi))

    v = hist_v[pl.ds(jstar * L, L)]
    rv = lax.rev(v, (0,))
    cum = jnp.cumsum(rv)
    m = (sbef + cum) >= r
    pc = jnp.max(plsc.all_reduce_population_count(m))
    d_star = L * jstar + pc - 1
    above = sbef + jnp.sum(jnp.where(m, 0, rv))
    return d_star, above


def _sc_body(key_hbm, out_hbm, slots_hbm, pf_hbm, pi_hbm, keys_v, hist_v,
             hist_b, tmp_v, vec_f, vec_i, gat_f, gat_i, btot_sm, sem):
    wid = lax.axis_index("s")
    base = wid * PER
    pltpu.sync_copy(key_hbm.at[pl.ds(base, PER)], keys_v)

    r = jnp.int32(K)
    d1, a1 = _hist_pass(keys_v, hist_v, hist_b, tmp_v, btot_sm, slots_hbm, sem, wid,
                        21, 0x7FF, None, None, r)
    r = r - a1
    p11 = d1
    d2, a2 = _hist_pass(keys_v, hist_v, hist_b, tmp_v, btot_sm, slots_hbm, sem, wid,
                        10, 0x7FF, 21, p11, r)
    r = r - a2
    p21 = (p11 << 11) | d2
    d3, _ = _hist_pass(keys_v, hist_v, hist_b, tmp_v, btot_sm, slots_hbm, sem, wid,
                       0, 0x3FF, 10, p21, r)
    thresh = lax.convert_element_type((p21 << 10) | d3, jnp.uint32)

    U = 4

    def _reduce(i, carry):
        accs = list(carry)
        for u in range(U):
            kv = keys_v[pl.ds((i * U + u) * L, L)]
            sign = kv >> jnp.uint32(31)
            flip = jnp.uint32(0xFFFFFFFF) ^ (sign * jnp.uint32(0x7FFFFFFF))
            lossv = plsc.bitcast(kv ^ flip, jnp.float32)
            m = kv >= thresh
            accs[2 * u] = accs[2 * u] + jnp.where(m, lossv, 0.0)
            accs[2 * u + 1] = accs[2 * u + 1] + jnp.where(m, 1, 0)
        return tuple(accs)

    init = []
    for u in range(U):
        init.append(jnp.zeros((L,), jnp.float32))
        init.append(jnp.zeros((L,), jnp.int32))
    accs = lax.fori_loop(0, NV // U, _reduce, tuple(init))
    acc_s = accs[0] + accs[2] + accs[4] + accs[6]
    acc_c = accs[1] + accs[3] + accs[5] + accs[7]
    vec_f[...] = acc_s
    vec_i[...] = acc_c
    pltpu.sync_copy(vec_f, pf_hbm.at[wid])
    pltpu.sync_copy(vec_i, pi_hbm.at[wid])
    plsc.subcore_barrier()
    pltpu.sync_copy(pf_hbm, gat_f)
    pltpu.sync_copy(pi_hbm, gat_i)
    tot_f = jnp.zeros((L,), jnp.float32)
    tot_i = jnp.zeros((L,), jnp.int32)
    for t in range(NT):
        tot_f = tot_f + gat_f[t]
        tot_i = tot_i + gat_i[t]
    s = jnp.sum(tot_f)
    cnt = lax.convert_element_type(jnp.sum(tot_i), jnp.float32)
    res = jnp.broadcast_to(s, (L,)) / (jnp.broadcast_to(cnt, (L,)) + EPS)
    vec_f[...] = res

    @pl.when(wid == 0)
    def _():
        pltpu.sync_copy(vec_f, out_hbm)


_sc_select = pl.kernel(
    _sc_body,
    out_type=(jax.ShapeDtypeStruct((L,), jnp.float32),
              jax.ShapeDtypeStruct((NT * NB * L,), jnp.int32),
              jax.ShapeDtypeStruct((NT, L), jnp.float32),
              jax.ShapeDtypeStruct((NT, L), jnp.int32)),
    mesh=plsc.VectorSubcoreMesh(core_axis_name="c", subcore_axis_name="s",
                                num_cores=1),
    compiler_params=pltpu.CompilerParams(needs_layout_passes=False),
    scratch_types=[
        pltpu.VMEM((PER,), jnp.uint32),
        pltpu.VMEM((NB * L,), jnp.int32),
        pltpu.VMEM((NB * L,), jnp.int32),
        pltpu.VMEM((NT * NB * L,), jnp.int32),
        pltpu.VMEM((L,), jnp.float32),
        pltpu.VMEM((L,), jnp.int32),
        pltpu.VMEM((NT, L), jnp.float32),
        pltpu.VMEM((NT, L), jnp.int32),
        pltpu.SMEM((NB,), jnp.int32),
        pltpu.SemaphoreType.DMA,
    ],
)


@jax.jit
def kernel(pred, target):
    keys = pl.pallas_call(
        _loss_keys_body,
        out_shape=jax.ShapeDtypeStruct((R, C), jnp.uint32),
    )(pred, target)
    out = _sc_select(keys.reshape(N))[0]
    return out[0]


# speedup vs baseline: 1.2553x; 1.0230x over previous
import jax
import jax.numpy as jnp
from jax import lax
from jax.experimental import pallas as pl
from jax.experimental.pallas import tpu as pltpu
from jax.experimental.pallas import tpu_sc as plsc

R, C = 32, 8192
N = R * C
K = N // 4
EPS = 1e-07
NT = 16
PER = N // NT
L = 16
NV = PER // L
NB = 128


def _loss_keys_body(pred_ref, targ_ref, key_ref):
    p = pred_ref[...]
    t = targ_ref[...]
    loss = jnp.maximum(p, 0.0) - p * t + jnp.log1p(jnp.exp(-jnp.abs(p)))
    bits = lax.bitcast_convert_type(loss, jnp.uint32)
    sign = bits >> jnp.uint32(31)
    flip = (jnp.uint32(0) - sign) | jnp.uint32(0x80000000)
    key_ref[...] = bits ^ flip


NW = NB * L


def _hist_pass(keys_v, hist_v, tmp_v, btot_sm, slots_hbm, sem, wid, shift,
               dmask, pshift, pref, r):
    zeros = jnp.zeros((L,), jnp.int32)
    ones = jnp.ones((L,), jnp.int32)

    @plsc.parallel_loop(0, NB, unroll=8)
    def _zero(j):
        hist_v[pl.ds(j * L, L)] = zeros

    @plsc.parallel_loop(0, NV, unroll=8)
    def _accum(i):
        kv = keys_v[pl.ds(i * L, L)]
        digit = lax.convert_element_type((kv >> jnp.uint32(shift))
                                         & jnp.uint32(dmask), jnp.int32)
        if pshift is None:
            plsc.addupdate_scatter(hist_v, [digit], ones)
        else:
            m = (kv >> jnp.uint32(pshift)) == jnp.uint32(pref)
            plsc.addupdate_scatter(hist_v, [digit], ones, mask=m)

    pltpu.sync_copy(hist_v, slots_hbm.at[pl.ds(wid * NW, NW)])
    plsc.subcore_barrier()
    pltpu.sync_copy(slots_hbm, tmp_v)
    plsc.subcore_barrier()

    @plsc.parallel_loop(0, NB, unroll=2)
    def _merge(j):
        acc = tmp_v[pl.ds(j * L, L)]
        for t in range(1, NT):
            acc = acc + tmp_v[pl.ds(t * NW + j * L, L)]
        hist_v[pl.ds(j * L, L)] = acc
        btot_sm[j] = jnp.sum(acc)

    def _scana(i, carry):
        s, found, jstar, sbef = carry
        j = NB - 1 - i
        t = btot_sm[j]
        cross = (found == 0) & ((s + t) >= r)
        jstar = jnp.where(cross, j, jstar)
        sbef = jnp.where(cross, s, sbef)
        found = found | cross.astype(jnp.int32)
        return s + t, found, jstar, sbef

    zi = jnp.int32(0)
    _, _, jstar, sbef = lax.fori_loop(0, NB, _scana, (zi, zi, zi, zi))

    v = hist_v[pl.ds(jstar * L, L)]
    rv = lax.rev(v, (0,))
    cum = jnp.cumsum(rv)
    m = (sbef + cum) >= r
    pc = jnp.max(plsc.all_reduce_population_count(m))
    d_star = L * jstar + pc - 1
    above = sbef + jnp.sum(jnp.where(m, 0, rv))
    return d_star, above


def _sc_body(key_hbm, out_hbm, slots_hbm, pf_hbm, pi_hbm, keys_v, hist_v,
             tmp_v, vec_f, vec_i, gat_f, gat_i, btot_sm, sem):
    wid = lax.axis_index("s")
    base = wid * PER
    pltpu.sync_copy(key_hbm.at[pl.ds(base, PER)], keys_v)

    r = jnp.int32(K)
    d1, a1 = _hist_pass(keys_v, hist_v, tmp_v, btot_sm, slots_hbm, sem, wid,
                        21, 0x7FF, None, None, r)
    r = r - a1
    p11 = d1
    d2, a2 = _hist_pass(keys_v, hist_v, tmp_v, btot_sm, slots_hbm, sem, wid,
                        10, 0x7FF, 21, p11, r)
    r = r - a2
    p21 = (p11 << 11) | d2
    d3, _ = _hist_pass(keys_v, hist_v, tmp_v, btot_sm, slots_hbm, sem, wid,
                       0, 0x3FF, 10, p21, r)
    thresh = lax.convert_element_type((p21 << 10) | d3, jnp.uint32)

    U = 4

    def _reduce(i, carry):
        accs = list(carry)
        for u in range(U):
            kv = keys_v[pl.ds((i * U + u) * L, L)]
            sign = kv >> jnp.uint32(31)
            flip = jnp.uint32(0xFFFFFFFF) ^ (sign * jnp.uint32(0x7FFFFFFF))
            lossv = plsc.bitcast(kv ^ flip, jnp.float32)
            m = kv >= thresh
            accs[2 * u] = accs[2 * u] + jnp.where(m, lossv, 0.0)
            accs[2 * u + 1] = accs[2 * u + 1] + jnp.where(m, 1, 0)
        return tuple(accs)

    init = []
    for u in range(U):
        init.append(jnp.zeros((L,), jnp.float32))
        init.append(jnp.zeros((L,), jnp.int32))
    accs = lax.fori_loop(0, NV // U, _reduce, tuple(init))
    acc_s = accs[0] + accs[2] + accs[4] + accs[6]
    acc_c = accs[1] + accs[3] + accs[5] + accs[7]
    vec_f[...] = acc_s
    vec_i[...] = acc_c
    pltpu.sync_copy(vec_f, pf_hbm.at[wid])
    pltpu.sync_copy(vec_i, pi_hbm.at[wid])
    plsc.subcore_barrier()
    pltpu.sync_copy(pf_hbm, gat_f)
    pltpu.sync_copy(pi_hbm, gat_i)
    tot_f = jnp.zeros((L,), jnp.float32)
    tot_i = jnp.zeros((L,), jnp.int32)
    for t in range(NT):
        tot_f = tot_f + gat_f[t]
        tot_i = tot_i + gat_i[t]
    s = jnp.sum(tot_f)
    cnt = lax.convert_element_type(jnp.sum(tot_i), jnp.float32)
    res = jnp.broadcast_to(s, (L,)) / (jnp.broadcast_to(cnt, (L,)) + EPS)
    vec_f[...] = res

    @pl.when(wid == 0)
    def _():
        pltpu.sync_copy(vec_f, out_hbm)


_sc_select = pl.kernel(
    _sc_body,
    out_type=(jax.ShapeDtypeStruct((L,), jnp.float32),
              jax.ShapeDtypeStruct((NT * NB * L,), jnp.int32),
              jax.ShapeDtypeStruct((NT, L), jnp.float32),
              jax.ShapeDtypeStruct((NT, L), jnp.int32)),
    mesh=plsc.VectorSubcoreMesh(core_axis_name="c", subcore_axis_name="s",
                                num_cores=1),
    compiler_params=pltpu.CompilerParams(needs_layout_passes=False),
    scratch_types=[
        pltpu.VMEM((PER,), jnp.uint32),
        pltpu.VMEM((NB * L,), jnp.int32),
        pltpu.VMEM((NT * NB * L,), jnp.int32),
        pltpu.VMEM((L,), jnp.float32),
        pltpu.VMEM((L,), jnp.int32),
        pltpu.VMEM((NT, L), jnp.float32),
        pltpu.VMEM((NT, L), jnp.int32),
        pltpu.SMEM((NB,), jnp.int32),
        pltpu.SemaphoreType.DMA,
    ],
)


@jax.jit
def kernel(pred, target):
    keys = pl.pallas_call(
        _loss_keys_body,
        out_shape=jax.ShapeDtypeStruct((R, C), jnp.uint32),
    )(pred, target)
    out = _sc_select(keys.reshape(N))[0]
    return out[0]


# speedup vs baseline: 1.2570x; 1.0013x over previous
import jax
import jax.numpy as jnp
from jax import lax
from jax.experimental import pallas as pl
from jax.experimental.pallas import tpu as pltpu
from jax.experimental.pallas import tpu_sc as plsc

R, C = 32, 8192
N = R * C
K = N // 4
EPS = 1e-07
NT = 16
PER = N // NT
L = 16
NV = PER // L
NB = 128


def _loss_keys_body(pred_ref, targ_ref, key_ref):
    p = pred_ref[...]
    t = targ_ref[...]
    loss = jnp.maximum(p, 0.0) - p * t + jnp.log1p(jnp.exp(-jnp.abs(p)))
    bits = lax.bitcast_convert_type(loss, jnp.uint32)
    sign = bits >> jnp.uint32(31)
    flip = (jnp.uint32(0) - sign) | jnp.uint32(0x80000000)
    key_ref[...] = bits ^ flip


NW = NB * L


def _hist_pass(keys_v, hist_v, tmp_v, btot_sm, slots_hbm, sem, wid, shift,
               dmask, pshift, pref, r):
    zeros = jnp.zeros((L,), jnp.int32)
    ones = jnp.ones((L,), jnp.int32)

    @plsc.parallel_loop(0, NB, unroll=8)
    def _zero(j):
        hist_v[pl.ds(j * L, L)] = zeros

    @plsc.parallel_loop(0, NV // 4, unroll=2)
    def _accum(i):
        for u in range(4):
            kv = keys_v[pl.ds((4 * i + u) * L, L)]
            digit = lax.convert_element_type((kv >> jnp.uint32(shift))
                                             & jnp.uint32(dmask), jnp.int32)
            if pshift is None:
                plsc.addupdate_scatter(hist_v, [digit], ones)
            else:
                m = (kv >> jnp.uint32(pshift)) == jnp.uint32(pref)
                plsc.addupdate_scatter(hist_v, [digit], ones, mask=m)

    pltpu.sync_copy(hist_v, slots_hbm.at[pl.ds(wid * NW, NW)])
    plsc.subcore_barrier()
    pltpu.sync_copy(slots_hbm, tmp_v)
    plsc.subcore_barrier()

    @plsc.parallel_loop(0, NB, unroll=2)
    def _merge(j):
        acc = tmp_v[pl.ds(j * L, L)]
        for t in range(1, NT):
            acc = acc + tmp_v[pl.ds(t * NW + j * L, L)]
        hist_v[pl.ds(j * L, L)] = acc
        btot_sm[j] = jnp.sum(acc)

    def _scana(i, carry):
        s, found, jstar, sbef = carry
        j = NB - 1 - i
        t = btot_sm[j]
        cross = (found == 0) & ((s + t) >= r)
        jstar = jnp.where(cross, j, jstar)
        sbef = jnp.where(cross, s, sbef)
        found = found | cross.astype(jnp.int32)
        return s + t, found, jstar, sbef

    zi = jnp.int32(0)
    _, _, jstar, sbef = lax.fori_loop(0, NB, _scana, (zi, zi, zi, zi))

    v = hist_v[pl.ds(jstar * L, L)]
    rv = lax.rev(v, (0,))
    cum = jnp.cumsum(rv)
    m = (sbef + cum) >= r
    pc = jnp.max(plsc.all_reduce_population_count(m))
    d_star = L * jstar + pc - 1
    above = sbef + jnp.sum(jnp.where(m, 0, rv))
    return d_star, above


def _sc_body(key_hbm, out_hbm, slots_hbm, pf_hbm, pi_hbm, keys_v, hist_v,
             tmp_v, vec_f, vec_i, gat_f, gat_i, btot_sm, sem):
    wid = lax.axis_index("s")
    base = wid * PER
    pltpu.sync_copy(key_hbm.at[pl.ds(base, PER)], keys_v)

    r = jnp.int32(K)
    d1, a1 = _hist_pass(keys_v, hist_v, tmp_v, btot_sm, slots_hbm, sem, wid,
                        21, 0x7FF, None, None, r)
    r = r - a1
    p11 = d1
    d2, a2 = _hist_pass(keys_v, hist_v, tmp_v, btot_sm, slots_hbm, sem, wid,
                        10, 0x7FF, 21, p11, r)
    r = r - a2
    p21 = (p11 << 11) | d2
    d3, _ = _hist_pass(keys_v, hist_v, tmp_v, btot_sm, slots_hbm, sem, wid,
                       0, 0x3FF, 10, p21, r)
    thresh = lax.convert_element_type((p21 << 10) | d3, jnp.uint32)

    U = 4

    def _reduce(i, carry):
        accs = list(carry)
        for u in range(U):
            kv = keys_v[pl.ds((i * U + u) * L, L)]
            sign = kv >> jnp.uint32(31)
            flip = jnp.uint32(0xFFFFFFFF) ^ (sign * jnp.uint32(0x7FFFFFFF))
            lossv = plsc.bitcast(kv ^ flip, jnp.float32)
            m = kv >= thresh
            accs[2 * u] = accs[2 * u] + jnp.where(m, lossv, 0.0)
            accs[2 * u + 1] = accs[2 * u + 1] + jnp.where(m, 1, 0)
        return tuple(accs)

    init = []
    for u in range(U):
        init.append(jnp.zeros((L,), jnp.float32))
        init.append(jnp.zeros((L,), jnp.int32))
    accs = lax.fori_loop(0, NV // U, _reduce, tuple(init))
    acc_s = accs[0] + accs[2] + accs[4] + accs[6]
    acc_c = accs[1] + accs[3] + accs[5] + accs[7]
    vec_f[...] = acc_s
    vec_i[...] = acc_c
    pltpu.sync_copy(vec_f, pf_hbm.at[wid])
    pltpu.sync_copy(vec_i, pi_hbm.at[wid])
    plsc.subcore_barrier()
    pltpu.sync_copy(pf_hbm, gat_f)
    pltpu.sync_copy(pi_hbm, gat_i)
    tot_f = jnp.zeros((L,), jnp.float32)
    tot_i = jnp.zeros((L,), jnp.int32)
    for t in range(NT):
        tot_f = tot_f + gat_f[t]
        tot_i = tot_i + gat_i[t]
    s = jnp.sum(tot_f)
    cnt = lax.convert_element_type(jnp.sum(tot_i), jnp.float32)
    res = jnp.broadcast_to(s, (L,)) / (jnp.broadcast_to(cnt, (L,)) + EPS)
    vec_f[...] = res

    @pl.when(wid == 0)
    def _():
        pltpu.sync_copy(vec_f, out_hbm)


_sc_select = pl.kernel(
    _sc_body,
    out_type=(jax.ShapeDtypeStruct((L,), jnp.float32),
              jax.ShapeDtypeStruct((NT * NB * L,), jnp.int32),
              jax.ShapeDtypeStruct((NT, L), jnp.float32),
              jax.ShapeDtypeStruct((NT, L), jnp.int32)),
    mesh=plsc.VectorSubcoreMesh(core_axis_name="c", subcore_axis_name="s",
                                num_cores=1),
    compiler_params=pltpu.CompilerParams(needs_layout_passes=False),
    scratch_types=[
        pltpu.VMEM((PER,), jnp.uint32),
        pltpu.VMEM((NB * L,), jnp.int32),
        pltpu.VMEM((NT * NB * L,), jnp.int32),
        pltpu.VMEM((L,), jnp.float32),
        pltpu.VMEM((L,), jnp.int32),
        pltpu.VMEM((NT, L), jnp.float32),
        pltpu.VMEM((NT, L), jnp.int32),
        pltpu.SMEM((NB,), jnp.int32),
        pltpu.SemaphoreType.DMA,
    ],
)


@jax.jit
def kernel(pred, target):
    keys = pl.pallas_call(
        _loss_keys_body,
        out_shape=jax.ShapeDtypeStruct((R, C), jnp.uint32),
    )(pred, target)
    out = _sc_select(keys.reshape(N))[0]
    return out[0]
